# baseline (speedup 1.0000x reference)
import functools
import math

import jax
import jax.numpy as jnp
from jax import lax
from jax.experimental import pallas as pl
from jax.experimental.pallas import tpu as pltpu
from jax.experimental.pallas import tpu_sc as plsc

EMB_SIZE = 64
SCALE = math.sqrt(EMB_SIZE)

NUM_CORES = 2
NUM_SUBCORES = 16
NUM_WORKERS = NUM_CORES * NUM_SUBCORES
LANES = 16

CB = 8


def _emb_body(n_batch, seq, n_chunks, tok_hbm, table_hbm, out_hbm, idx_v,
              rows_v, sem0, sem1):
  sems = (sem0, sem1)
  bpw = n_batch // NUM_WORKERS
  wid = lax.axis_index("s") * NUM_CORES + lax.axis_index("c")
  b0 = wid * bpw

  pltpu.sync_copy(tok_hbm.at[pl.ds(b0, bpw), :], idx_v)

  def gather_descs(k, b):
    return [
        pltpu.make_async_copy(
            table_hbm.at[idx_v.at[k * CB + j]],
            rows_v.at[b, j],
            sems[b],
        )
        for j in range(CB)
    ]

  def issue(k, b):
    for d in gather_descs(k, b):
      d.start()

  issue(0, 0)

  @pl.loop(0, n_chunks // 2)
  def _(k2):
    for b in range(2):
      k = k2 * 2 + b

      @pl.when(k + 1 < n_chunks)
      def _():
        issue(k + 1, 1 - b)

      for d in gather_descs(k, b):
        d.wait()

      for j in range(CB):

        @pl.loop(0, seq)
        def _(i):
          for jj in range(EMB_SIZE // LANES):
            sl = pl.ds(jj * LANES, LANES)
            rows_v[b, j, i, sl] = rows_v[b, j, i, sl] * SCALE

      pltpu.sync_copy(
          rows_v.at[b], out_hbm.at[pl.ds(b0 + k * CB, CB), :, :]
      )


@jax.jit
def _emb_call(tokens, table):
  n_batch, seq = tokens.shape
  n_chunks = n_batch // NUM_WORKERS // CB
  mesh = plsc.VectorSubcoreMesh(
      core_axis_name="c", subcore_axis_name="s", num_cores=NUM_CORES
  )
  return pl.kernel(
      functools.partial(_emb_body, n_batch, seq, n_chunks),
      out_type=jax.ShapeDtypeStruct((n_batch, seq, EMB_SIZE), jnp.float32),
      mesh=mesh,
      scratch_types=[
          pltpu.VMEM((n_batch // NUM_WORKERS, seq), jnp.int32),
          pltpu.VMEM((2, CB, seq, EMB_SIZE), jnp.float32),
          pltpu.SemaphoreType.DMA,
          pltpu.SemaphoreType.DMA,
      ],
      compiler_params=pltpu.CompilerParams(use_tc_tiling_on_sc=False),
  )(tokens, table)


def kernel(tokens, embedding_weight):
  n_batch, seq = tokens.shape
  assert n_batch % (NUM_WORKERS * CB * 2) == 0
  return _emb_call(tokens.astype(jnp.int32), embedding_weight)
